# baseline (speedup 1.0000x reference)
import functools

import jax
import jax.numpy as jnp
import numpy as np
from jax import lax
from jax.experimental import pallas as pl
from jax.experimental.pallas import tpu as pltpu
from jax.experimental.pallas import tpu_sc as plsc

_N = 1_000_000
_D = 16
_NZ = 500_000
_NC = 2
_NS = 16
_NW = _NC * _NS
_S = _N // _NW
_CH = 128
_CCH = 1250
_NCOPY = _S // _CCH
_RING = 5


@functools.cache
def _build_index_table():
    idx = np.sort(np.asarray(jax.random.permutation(jax.random.key(42), _N)[:_NZ]))
    parts = [idx[(idx >= w * _S) & (idx < (w + 1) * _S)] for w in range(_NW)]
    assert all(len(p) > 0 for p in parts)
    n_chunks = max(-(-len(p) // _CH) for p in parts)
    k = n_chunks * _CH
    tab = np.empty((_NW, k), np.int32)
    for w, p in enumerate(parts):
        tab[w, : len(p)] = p
        tab[w, len(p):] = p[-1]
    return tab.reshape(_NW, n_chunks, _CH)


try:
    _N_CHUNKS = _build_index_table().shape[1]
except Exception:
    _N_CHUNKS = None


@functools.cache
def _get_sc_kernel():
    n_chunks = _build_index_table().shape[1]
    mesh = plsc.VectorSubcoreMesh(
        core_axis_name="c", subcore_axis_name="s", num_cores=_NC, num_subcores=_NS
    )

    @functools.partial(
        pl.kernel,
        out_type=(
            jax.ShapeDtypeStruct((_N, _D), jnp.float32),
            jax.ShapeDtypeStruct((_N, _D), jnp.float32),
        ),
        mesh=mesh,
        compiler_params=pltpu.CompilerParams(use_tc_tiling_on_sc=False),
        scratch_types=(
            [pltpu.VMEM((_CCH, _D), jnp.float32) for _ in range(_RING)]
            + [
                pltpu.VMEM((n_chunks, _CH), jnp.int32),
                pltpu.VMEM((_CH, _D), jnp.float32),
            ]
            + [pltpu.SemaphoreType.DMA for _ in range(2 * _RING + 2)]
        ),
    )
    def _sc_dropout(emb1, emb2, idx_hbm, zeros_hbm, out1, out2, *scratch):
        bufs = scratch[:_RING]
        idx_v = scratch[_RING]
        zeros_v = scratch[_RING + 1]
        insems = scratch[_RING + 2:2 * _RING + 2]
        outsems = scratch[2 * _RING + 2:3 * _RING + 2]
        lsem = scratch[3 * _RING + 2]
        ssem = scratch[3 * _RING + 3]

        c = lax.axis_index("c")
        s = lax.axis_index("s")
        wid = s * _NC + c
        base = wid * _S

        pltpu.async_copy(zeros_hbm, zeros_v, lsem).wait()
        pltpu.async_copy(idx_hbm.at[wid], idx_v, lsem).wait()

        for src, dst in ((emb1, out1), (emb2, out2)):
            for b in range(_RING):
                pltpu.async_copy(
                    src.at[pl.ds(base + b * _CCH, _CCH)], bufs[b], insems[b]
                )

            @pl.loop(0, _NCOPY // _RING)
            def _copy(jg):
                j0 = jg * _RING
                for b in range(_RING):
                    j = j0 + b
                    row = base + j * _CCH
                    pltpu.make_async_copy(
                        src.at[pl.ds(row, _CCH)], bufs[b], insems[b]
                    ).wait()
                    pltpu.async_copy(
                        bufs[b], dst.at[pl.ds(row, _CCH)], outsems[b]
                    )
                    pltpu.make_async_copy(
                        bufs[b], dst.at[pl.ds(row, _CCH)], outsems[b]
                    ).wait()

                    @pl.when(jg < _NCOPY // _RING - 1)
                    def _prefetch():
                        pltpu.async_copy(
                            src.at[pl.ds(row + _RING * _CCH, _CCH)],
                            bufs[b],
                            insems[b],
                        )

        @pl.loop(0, n_chunks)
        def _fire(j):
            pltpu.async_copy(zeros_v, out1.at[idx_v.at[j]], ssem)
            pltpu.async_copy(zeros_v, out2.at[idx_v.at[j]], ssem)

        @pl.loop(0, n_chunks)
        def _drain(j):
            pltpu.make_async_copy(zeros_v, out1.at[idx_v.at[j]], ssem).wait()
            pltpu.make_async_copy(zeros_v, out2.at[idx_v.at[j]], ssem).wait()

    return _sc_dropout


def kernel(emb1, emb2):
    idx_tab = jnp.asarray(_build_index_table())
    zeros = jnp.zeros((_CH, _D), jnp.float32)
    return _get_sc_kernel()(emb1, emb2, idx_tab, zeros)
